# baseline (speedup 1.0000x reference)
import functools

import jax
import jax.numpy as jnp
from jax import lax
from jax.experimental import pallas as pl
from jax.experimental.pallas import tpu as pltpu
from jax.experimental.pallas import tpu_sc as plsc

N = 2560
E = 40960
H = 128
D_OUT = 40

NCORE = 2
NSUB = 16
LANES = 16
CHUNK = 8192


def _prep_body(x_ref, w_ref, as_ref, ad_ref, ht_ref, als_ref, ald_ref):
    xv = x_ref[...]
    wv = w_ref[...]
    ht = lax.dot_general(wv, xv, (((0,), (1,)), ((), ())),
                         preferred_element_type=jnp.float32)
    ht_ref[...] = ht
    als_ref[...] = lax.dot_general(as_ref[...], ht, (((1,), (0,)), ((), ())),
                                   preferred_element_type=jnp.float32)
    ald_ref[...] = lax.dot_general(ad_ref[...], ht, (((1,), (0,)), ((), ())),
                                   preferred_element_type=jnp.float32)


def _tc_prep(x, w, a_s, a_d):
    return pl.pallas_call(
        _prep_body,
        out_shape=(
            jax.ShapeDtypeStruct((H, N), jnp.float32),
            jax.ShapeDtypeStruct((1, N), jnp.float32),
            jax.ShapeDtypeStruct((1, N), jnp.float32),
        ),
    )(x, w, a_s, a_d)


def _bn_relu(y, g, be):
    m = jnp.mean(y, axis=1, keepdims=True)
    c = y - m
    v = jnp.mean(c * c, axis=1, keepdims=True)
    return jnp.maximum(c * lax.rsqrt(v + 1e-5) * g + be, 0.0)


def _mid_body(ot_ref, b_ref, g_ref, be_ref, w_ref, as_ref, ad_ref,
              ht_ref, als_ref, ald_ref):
    y = ot_ref[...] + b_ref[...]
    h = _bn_relu(y, g_ref[...], be_ref[...])
    ht = lax.dot_general(w_ref[...], h, (((0,), (0,)), ((), ())),
                         preferred_element_type=jnp.float32)
    ht_ref[...] = ht
    als_ref[...] = lax.dot_general(as_ref[...], ht, (((1,), (0,)), ((), ())),
                                   preferred_element_type=jnp.float32)
    ald_ref[...] = lax.dot_general(ad_ref[...], ht, (((1,), (0,)), ((), ())),
                                   preferred_element_type=jnp.float32)


def _tc_mid(ot, b, g, be, w, a_s, a_d):
    return pl.pallas_call(
        _mid_body,
        out_shape=(
            jax.ShapeDtypeStruct((H, N), jnp.float32),
            jax.ShapeDtypeStruct((1, N), jnp.float32),
            jax.ShapeDtypeStruct((1, N), jnp.float32),
        ),
    )(ot, b, g, be, w, a_s, a_d)


def _final_body(ot_ref, b_ref, g_ref, be_ref, wf_ref, bf_ref, mask_ref,
                adj_ref, out_ref):
    y = ot_ref[...] + b_ref[...]
    h = _bn_relu(y, g_ref[...], be_ref[...])
    o = lax.dot_general(h, wf_ref[...], (((0,), (0,)), ((), ())),
                        preferred_element_type=jnp.float32)
    o = (o + bf_ref[...]) * mask_ref[...]
    out_ref[...] = o + adj_ref[...]


def _tc_final(ot, b, g, be, wf, bf, mask, adj2d):
    return pl.pallas_call(
        _final_body,
        out_shape=jax.ShapeDtypeStruct((N, D_OUT), jnp.float32),
    )(ot, b, g, be, wf, bf, mask, adj2d)


def _sc_layer_body(ht_hbm, als_hbm, ald_hbm, px_hbm, py_hbm, pz_hbm,
                   src_hbm, dst_hbm, out_hbm, alpha_hbm,
                   srcv, dstv, alsv, aldv, pxv, pyv, pzv,
                   exbuf, dpriv, denl, abuf, acc, tmpall,
                   h0, h1, h2, h3, o0, o1, o2, o3,
                   srcc, dstc, alpc, srcc2, dstc2, alpc2,
                   sem0, sem1, sem2,
                   sh_all, sh_den):
    core = lax.axis_index("c")
    sub = lax.axis_index("s")
    eb = E // NSUB
    base = sub * eb
    cw = N // NSUB

    zero16 = jnp.zeros((LANES,), jnp.float32)
    ch = (core * NSUB + sub) * 4
    nch = E // CHUNK
    bufs = [(srcc, dstc, alpc), (srcc2, dstc2, alpc2)]
    hs_refs = (h0, h1, h2, h3)
    os_refs = (o0, o1, o2, o3)

    pre2 = [
        pltpu.async_copy(ht_hbm.at[ch], h0, sem0),
        pltpu.async_copy(ht_hbm.at[ch + 1], h1, sem0),
        pltpu.async_copy(ht_hbm.at[ch + 2], h2, sem0),
        pltpu.async_copy(ht_hbm.at[ch + 3], h3, sem0),
        pltpu.async_copy(src_hbm.at[pl.ds(0, CHUNK)], srcc, sem0),
        pltpu.async_copy(dst_hbm.at[pl.ds(0, CHUNK)], dstc, sem0),
    ]
    scope1 = jax.named_scope("p1_edge")
    scope1.__enter__()
    pre1 = [
        pltpu.async_copy(src_hbm.at[pl.ds(base, eb)], srcv, sem1),
        pltpu.async_copy(dst_hbm.at[pl.ds(base, eb)], dstv, sem1),
        pltpu.async_copy(als_hbm, alsv, sem1),
        pltpu.async_copy(ald_hbm, aldv, sem1),
        pltpu.async_copy(px_hbm, pxv, sem1),
        pltpu.async_copy(py_hbm, pyv, sem1),
        pltpu.async_copy(pz_hbm, pzv, sem1),
    ]

    def zero_body(i, _):
        sl = pl.ds(i * LANES, LANES)
        dpriv[sl] = zero16
        o0[sl] = zero16
        o1[sl] = zero16
        o2[sl] = zero16
        o3[sl] = zero16
        return 0

    lax.fori_loop(0, N // LANES, zero_body, 0)
    for c in pre1:
        c.wait()

    U1 = 4

    def p1_step(off):
        sv = srcv[pl.ds(off, LANES)]
        dv = dstv[pl.ds(off, LANES)]
        ax = plsc.load_gather(pxv, [sv]) - plsc.load_gather(pxv, [dv])
        ay = plsc.load_gather(pyv, [sv]) - plsc.load_gather(pyv, [dv])
        az = plsc.load_gather(pzv, [sv]) - plsc.load_gather(pzv, [dv])
        s2 = ax * ax + ay * ay + az * az
        ib = jnp.int32(0x5F3759DF) - (plsc.bitcast(s2, jnp.int32) >> 1)
        y = plsc.bitcast(ib, jnp.float32)
        hs = 0.5 * s2
        y = y * (1.5 - (hs * y) * y)
        y = y * (1.5 - (hs * y) * y)
        y = y * (1.5 - (hs * y) * y)
        dist = s2 * y
        ew = 1.0 / (dist + 1e-6)
        e = plsc.load_gather(alsv, [sv]) + plsc.load_gather(aldv, [dv])
        e = jnp.maximum(e, 0.2 * e)
        return dv, jnp.exp(e) * ew

    def p1_body(i, _):
        off = i * (U1 * LANES)
        res = [p1_step(off + u * LANES) for u in range(U1)]
        for u in range(U1):
            dv, ex = res[u]
            exbuf[pl.ds(off + u * LANES, LANES)] = ex
            plsc.addupdate_scatter(dpriv, [dv], ex)
        return 0

    lax.fori_loop(0, eb // (U1 * LANES), p1_body, 0)
    scope1.__exit__(None, None, None)

    scope2 = jax.named_scope("p1_combine")
    scope2.__enter__()
    pltpu.sync_copy(dpriv, sh_all.at[pl.ds(sub * N, N)])
    plsc.subcore_barrier()

    reads = [pltpu.async_copy(sh_all.at[pl.ds(r * N + sub * cw, cw)],
                              tmpall.at[pl.ds(r * cw, cw)], sem2)
             for r in range(NSUB)]
    for c in reads:
        c.wait()

    def comb_body(i, _):
        sl = pl.ds(i * LANES, LANES)
        s = tmpall[sl]
        for r in range(1, NSUB):
            s = s + tmpall[pl.ds(r * cw + i * LANES, LANES)]
        acc[sl] = s
        return 0

    lax.fori_loop(0, cw // LANES, comb_body, 0)
    pltpu.sync_copy(acc, sh_den.at[pl.ds(sub * cw, cw)])
    plsc.subcore_barrier()
    pltpu.sync_copy(sh_den, denl)

    def p1b_body(i, _):
        off = i * (U1 * LANES)
        dvs = [dstv[pl.ds(off + u * LANES, LANES)] for u in range(U1)]
        dens = [plsc.load_gather(denl, [dvs[u]]) for u in range(U1)]
        for u in range(U1):
            sl = pl.ds(off + u * LANES, LANES)
            abuf[sl] = exbuf[sl] / (dens[u] + 1e-16)
        return 0

    lax.fori_loop(0, eb // (U1 * LANES), p1b_body, 0)
    pltpu.sync_copy(abuf, alpha_hbm.at[pl.ds(base, eb)])
    plsc.subcore_barrier()
    scope2.__exit__(None, None, None)

    scope3 = jax.named_scope("p2_agg")
    scope3.__enter__()
    for c in pre2:
        c.wait()

    def start_chunk(k, b):
        sb, db, ab = bufs[b]
        c1 = pltpu.async_copy(src_hbm.at[pl.ds(k * CHUNK, CHUNK)], sb, sem0)
        c2 = pltpu.async_copy(dst_hbm.at[pl.ds(k * CHUNK, CHUNK)], db, sem1)
        c3 = pltpu.async_copy(alpha_hbm.at[pl.ds(k * CHUNK, CHUNK)], ab, sem2)
        return (c1, c2, c3)

    U = 4
    pending = [pltpu.async_copy(alpha_hbm.at[pl.ds(0, CHUNK)], alpc, sem2)]
    for k in range(nch):
        for c in pending:
            c.wait()
        if k + 1 < nch:
            pending = start_chunk(k + 1, (k + 1) % 2)
        sb, db, ab = bufs[k % 2]

        def p2_body(i, _):
            off = i * (U * LANES)
            svs = [sb[pl.ds(off + u * LANES, LANES)] for u in range(U)]
            dvs = [db[pl.ds(off + u * LANES, LANES)] for u in range(U)]
            avs = [ab[pl.ds(off + u * LANES, LANES)] for u in range(U)]
            gs = [[plsc.load_gather(h, [svs[u]]) for h in hs_refs]
                  for u in range(U)]
            ms = [[gs[u][j] * avs[u] for j in range(4)] for u in range(U)]
            for u in range(U):
                for j in range(4):
                    plsc.addupdate_scatter(os_refs[j], [dvs[u]], ms[u][j])
            return 0

        lax.fori_loop(0, CHUNK // (U * LANES), p2_body, 0)

    pltpu.sync_copy(o0, out_hbm.at[ch])
    pltpu.sync_copy(o1, out_hbm.at[ch + 1])
    pltpu.sync_copy(o2, out_hbm.at[ch + 2])
    pltpu.sync_copy(o3, out_hbm.at[ch + 3])
    scope3.__exit__(None, None, None)


@functools.partial(jax.jit, static_argnames=())
def _sc_layer(ht, als, ald, px, py, pz, src, dst):
    f32 = jnp.float32
    mesh = plsc.VectorSubcoreMesh(core_axis_name="c", subcore_axis_name="s",
                                  num_cores=NCORE, num_subcores=NSUB)
    return pl.kernel(
        _sc_layer_body,
        out_type=(jax.ShapeDtypeStruct((H, N), f32),
                  jax.ShapeDtypeStruct((E,), f32)),
        mesh=mesh,
        compiler_params=pltpu.CompilerParams(needs_layout_passes=False),
        scratch_types=[
            pltpu.VMEM((E // NSUB,), jnp.int32),
            pltpu.VMEM((E // NSUB,), jnp.int32),
            pltpu.VMEM((N,), f32),
            pltpu.VMEM((N,), f32),
            pltpu.VMEM((N,), f32),
            pltpu.VMEM((N,), f32),
            pltpu.VMEM((N,), f32),
            pltpu.VMEM((E // NSUB,), f32),
            pltpu.VMEM((N,), f32),
            pltpu.VMEM((N,), f32),
            pltpu.VMEM((E // NSUB,), f32),
            pltpu.VMEM((N // NSUB,), f32),
            pltpu.VMEM((N,), f32),
            pltpu.VMEM((N,), f32),
            pltpu.VMEM((N,), f32),
            pltpu.VMEM((N,), f32),
            pltpu.VMEM((N,), f32),
            pltpu.VMEM((N,), f32),
            pltpu.VMEM((N,), f32),
            pltpu.VMEM((N,), f32),
            pltpu.VMEM((N,), f32),
            pltpu.VMEM((CHUNK,), jnp.int32),
            pltpu.VMEM((CHUNK,), jnp.int32),
            pltpu.VMEM((CHUNK,), f32),
            pltpu.VMEM((CHUNK,), jnp.int32),
            pltpu.VMEM((CHUNK,), jnp.int32),
            pltpu.VMEM((CHUNK,), f32),
            pltpu.SemaphoreType.DMA,
            pltpu.SemaphoreType.DMA,
            pltpu.SemaphoreType.DMA,
            pltpu.VMEM_SHARED((NSUB * N,), f32),
            pltpu.VMEM_SHARED((N,), f32),
        ],
    )(ht, als, ald, px, py, pz, src, dst)


def kernel(x, edge_index, pos, mask, adjacency, W1, a1s, a1d, b1, g1, be1,
           W2, a2s, a2d, b2, g2, be2, Wf, bf):
    src = edge_index[0]
    dst = edge_index[1]
    px = pos[:, 0]
    py = pos[:, 1]
    pz = pos[:, 2]

    ht1, als1, ald1 = _tc_prep(x, W1, a1s, a1d)
    ot1, _unused1 = _sc_layer(ht1, als1.reshape(N), ald1.reshape(N),
                              px, py, pz, src, dst)
    ht2, als2, ald2 = _tc_mid(ot1, b1.reshape(H, 1), g1.reshape(H, 1),
                              be1.reshape(H, 1), W2, a2s, a2d)
    ot2, _unused2 = _sc_layer(ht2, als2.reshape(N), ald2.reshape(N),
                              px, py, pz, src, dst)
    out2d = _tc_final(ot2, b2.reshape(H, 1), g2.reshape(H, 1),
                      be2.reshape(H, 1), Wf, bf.reshape(1, D_OUT),
                      mask.reshape(N, 1), adjacency.reshape(N, D_OUT))
    return out2d.reshape(64, 40, 40)


# speedup vs baseline: 1.0373x; 1.0373x over previous
import functools

import jax
import jax.numpy as jnp
from jax import lax
from jax.experimental import pallas as pl
from jax.experimental.pallas import tpu as pltpu
from jax.experimental.pallas import tpu_sc as plsc

N = 2560
E = 40960
H = 128
D_OUT = 40

NCORE = 2
NSUB = 16
LANES = 16
CHUNK = 8192


def _prep_body(x_ref, w_ref, as_ref, ad_ref, ht_ref, als_ref, ald_ref):
    xv = x_ref[...]
    wv = w_ref[...]
    ht = lax.dot_general(wv, xv, (((0,), (1,)), ((), ())),
                         preferred_element_type=jnp.float32)
    ht_ref[...] = ht
    als_ref[...] = lax.dot_general(as_ref[...], ht, (((1,), (0,)), ((), ())),
                                   preferred_element_type=jnp.float32)
    ald_ref[...] = lax.dot_general(ad_ref[...], ht, (((1,), (0,)), ((), ())),
                                   preferred_element_type=jnp.float32)


def _tc_prep(x, w, a_s, a_d):
    return pl.pallas_call(
        _prep_body,
        out_shape=(
            jax.ShapeDtypeStruct((H, N), jnp.float32),
            jax.ShapeDtypeStruct((1, N), jnp.float32),
            jax.ShapeDtypeStruct((1, N), jnp.float32),
        ),
    )(x, w, a_s, a_d)


def _bn_relu(y, g, be):
    m = jnp.mean(y, axis=1, keepdims=True)
    c = y - m
    v = jnp.mean(c * c, axis=1, keepdims=True)
    return jnp.maximum(c * lax.rsqrt(v + 1e-5) * g + be, 0.0)


def _mid_body(ot_ref, b_ref, g_ref, be_ref, w_ref, as_ref, ad_ref,
              ht_ref, als_ref, ald_ref):
    y = ot_ref[...] + b_ref[...]
    h = _bn_relu(y, g_ref[...], be_ref[...])
    ht = lax.dot_general(w_ref[...], h, (((0,), (0,)), ((), ())),
                         preferred_element_type=jnp.float32)
    ht_ref[...] = ht
    als_ref[...] = lax.dot_general(as_ref[...], ht, (((1,), (0,)), ((), ())),
                                   preferred_element_type=jnp.float32)
    ald_ref[...] = lax.dot_general(ad_ref[...], ht, (((1,), (0,)), ((), ())),
                                   preferred_element_type=jnp.float32)


def _tc_mid(ot, b, g, be, w, a_s, a_d):
    return pl.pallas_call(
        _mid_body,
        out_shape=(
            jax.ShapeDtypeStruct((H, N), jnp.float32),
            jax.ShapeDtypeStruct((1, N), jnp.float32),
            jax.ShapeDtypeStruct((1, N), jnp.float32),
        ),
    )(ot, b, g, be, w, a_s, a_d)


def _final_body(ot_ref, b_ref, g_ref, be_ref, wf_ref, bf_ref, mask_ref,
                adj_ref, out_ref):
    y = ot_ref[...] + b_ref[...]
    h = _bn_relu(y, g_ref[...], be_ref[...])
    o = lax.dot_general(h, wf_ref[...], (((0,), (0,)), ((), ())),
                        preferred_element_type=jnp.float32)
    o = (o + bf_ref[...]) * mask_ref[...]
    out_ref[...] = o + adj_ref[...]


def _tc_final(ot, b, g, be, wf, bf, mask, adj2d):
    return pl.pallas_call(
        _final_body,
        out_shape=jax.ShapeDtypeStruct((N, D_OUT), jnp.float32),
    )(ot, b, g, be, wf, bf, mask, adj2d)


def _sc_layer_body(ht_hbm, als_hbm, ald_hbm, px_hbm, py_hbm, pz_hbm,
                   src_hbm, dst_hbm, out_hbm,
                   srcv, dstv, alsv, aldv, pxv, pyv, pzv,
                   exbuf, dpriv, denl, acc, tmpall,
                   h0, h1, h2, h3, o0, o1, o2, o3,
                   srcc, dstc, alpc, srcc2, dstc2, alpc2,
                   sem0, sem1, sem2, sem3, sem4,
                   sh_all, sh_den, sh_ex):
    core = lax.axis_index("c")
    sub = lax.axis_index("s")
    eb = E // NSUB
    base = sub * eb
    cw = N // NSUB

    zero16 = jnp.zeros((LANES,), jnp.float32)
    ch = (core * NSUB + sub) * 4
    nch = E // CHUNK
    bufs = [(srcc, dstc, alpc), (srcc2, dstc2, alpc2)]
    hs_refs = (h0, h1, h2, h3)
    os_refs = (o0, o1, o2, o3)

    pre2 = [
        pltpu.async_copy(ht_hbm.at[ch], h0, sem0),
        pltpu.async_copy(ht_hbm.at[ch + 1], h1, sem0),
        pltpu.async_copy(ht_hbm.at[ch + 2], h2, sem0),
        pltpu.async_copy(ht_hbm.at[ch + 3], h3, sem0),
        pltpu.async_copy(src_hbm.at[pl.ds(0, CHUNK)], srcc, sem0),
        pltpu.async_copy(dst_hbm.at[pl.ds(0, CHUNK)], dstc, sem0),
    ]
    scope1 = jax.named_scope("p1_edge")
    scope1.__enter__()
    pre1 = [
        pltpu.async_copy(src_hbm.at[pl.ds(base, eb)], srcv, sem1),
        pltpu.async_copy(dst_hbm.at[pl.ds(base, eb)], dstv, sem1),
        pltpu.async_copy(als_hbm, alsv, sem1),
        pltpu.async_copy(ald_hbm, aldv, sem1),
        pltpu.async_copy(px_hbm, pxv, sem1),
        pltpu.async_copy(py_hbm, pyv, sem1),
        pltpu.async_copy(pz_hbm, pzv, sem1),
    ]

    def zero_body(i, _):
        sl = pl.ds(i * LANES, LANES)
        dpriv[sl] = zero16
        o0[sl] = zero16
        o1[sl] = zero16
        o2[sl] = zero16
        o3[sl] = zero16
        return 0

    lax.fori_loop(0, N // LANES, zero_body, 0)
    for c in pre1:
        c.wait()

    U1 = 4

    def p1_step(off):
        sv = srcv[pl.ds(off, LANES)]
        dv = dstv[pl.ds(off, LANES)]
        ax = plsc.load_gather(pxv, [sv]) - plsc.load_gather(pxv, [dv])
        ay = plsc.load_gather(pyv, [sv]) - plsc.load_gather(pyv, [dv])
        az = plsc.load_gather(pzv, [sv]) - plsc.load_gather(pzv, [dv])
        s2 = ax * ax + ay * ay + az * az
        ib = jnp.int32(0x5F3759DF) - (plsc.bitcast(s2, jnp.int32) >> 1)
        y = plsc.bitcast(ib, jnp.float32)
        hs = 0.5 * s2
        y = y * (1.5 - (hs * y) * y)
        y = y * (1.5 - (hs * y) * y)
        y = y * (1.5 - (hs * y) * y)
        dist = s2 * y
        ew = 1.0 / (dist + 1e-6)
        e = plsc.load_gather(alsv, [sv]) + plsc.load_gather(aldv, [dv])
        e = jnp.maximum(e, 0.2 * e)
        return dv, jnp.exp(e) * ew

    def p1_body(i, _):
        off = i * (U1 * LANES)
        res = [p1_step(off + u * LANES) for u in range(U1)]
        for u in range(U1):
            dv, ex = res[u]
            exbuf[pl.ds(off + u * LANES, LANES)] = ex
            plsc.addupdate_scatter(dpriv, [dv], ex)
        return 0

    lax.fori_loop(0, eb // (U1 * LANES), p1_body, 0)
    scope1.__exit__(None, None, None)

    scope2 = jax.named_scope("p1_combine")
    scope2.__enter__()
    pltpu.sync_copy(exbuf, sh_ex.at[pl.ds(base, eb)])
    pltpu.sync_copy(dpriv, sh_all.at[pl.ds(sub * N, N)])
    plsc.subcore_barrier()

    pending = [pltpu.async_copy(sh_ex.at[pl.ds(0, CHUNK)], alpc, sem2)]
    reads = [pltpu.async_copy(sh_all.at[pl.ds(r * N + sub * cw, cw)],
                              tmpall.at[pl.ds(r * cw, cw)], sem3)
             for r in range(NSUB)]
    for c in reads:
        c.wait()

    def comb_body(i, _):
        sl = pl.ds(i * LANES, LANES)
        s = tmpall[sl]
        for r in range(1, NSUB):
            s = s + tmpall[pl.ds(r * cw + i * LANES, LANES)]
        acc[sl] = s
        return 0

    lax.fori_loop(0, cw // LANES, comb_body, 0)
    pltpu.sync_copy(acc, sh_den.at[pl.ds(sub * cw, cw)])
    plsc.subcore_barrier()
    den_read = pltpu.async_copy(sh_den, denl, sem4)
    scope2.__exit__(None, None, None)

    scope3 = jax.named_scope("p2_agg")
    scope3.__enter__()
    for c in pre2:
        c.wait()

    def start_chunk(k, b):
        sb, db, ab = bufs[b]
        c1 = pltpu.async_copy(src_hbm.at[pl.ds(k * CHUNK, CHUNK)], sb, sem0)
        c2 = pltpu.async_copy(dst_hbm.at[pl.ds(k * CHUNK, CHUNK)], db, sem1)
        c3 = pltpu.async_copy(sh_ex.at[pl.ds(k * CHUNK, CHUNK)], ab, sem2)
        return (c1, c2, c3)

    U = 4
    for k in range(nch):
        for c in pending:
            c.wait()
        if k + 1 < nch:
            pending = start_chunk(k + 1, (k + 1) % 2)
        sb, db, ab = bufs[k % 2]

        def p2_body(i, _):
            off = i * (U * LANES)
            svs = [sb[pl.ds(off + u * LANES, LANES)] for u in range(U)]
            dvs = [db[pl.ds(off + u * LANES, LANES)] for u in range(U)]
            avs = [ab[pl.ds(off + u * LANES, LANES)] for u in range(U)]
            gs = [[plsc.load_gather(h, [svs[u]]) for h in hs_refs]
                  for u in range(U)]
            ms = [[gs[u][j] * avs[u] for j in range(4)] for u in range(U)]
            for u in range(U):
                for j in range(4):
                    plsc.addupdate_scatter(os_refs[j], [dvs[u]], ms[u][j])
            return 0

        lax.fori_loop(0, CHUNK // (U * LANES), p2_body, 0)

    den_read.wait()

    def norm_body(i, _):
        sl = pl.ds(i * LANES, LANES)
        r = 1.0 / (denl[sl] + 1e-16)
        o0[sl] = o0[sl] * r
        o1[sl] = o1[sl] * r
        o2[sl] = o2[sl] * r
        o3[sl] = o3[sl] * r
        return 0

    lax.fori_loop(0, N // LANES, norm_body, 0)

    pltpu.sync_copy(o0, out_hbm.at[ch])
    pltpu.sync_copy(o1, out_hbm.at[ch + 1])
    pltpu.sync_copy(o2, out_hbm.at[ch + 2])
    pltpu.sync_copy(o3, out_hbm.at[ch + 3])
    scope3.__exit__(None, None, None)


@functools.partial(jax.jit, static_argnames=())
def _sc_layer(ht, als, ald, px, py, pz, src, dst):
    f32 = jnp.float32
    mesh = plsc.VectorSubcoreMesh(core_axis_name="c", subcore_axis_name="s",
                                  num_cores=NCORE, num_subcores=NSUB)
    return pl.kernel(
        _sc_layer_body,
        out_type=jax.ShapeDtypeStruct((H, N), f32),
        mesh=mesh,
        compiler_params=pltpu.CompilerParams(needs_layout_passes=False),
        scratch_types=[
            pltpu.VMEM((E // NSUB,), jnp.int32),
            pltpu.VMEM((E // NSUB,), jnp.int32),
            pltpu.VMEM((N,), f32),
            pltpu.VMEM((N,), f32),
            pltpu.VMEM((N,), f32),
            pltpu.VMEM((N,), f32),
            pltpu.VMEM((N,), f32),
            pltpu.VMEM((E // NSUB,), f32),
            pltpu.VMEM((N,), f32),
            pltpu.VMEM((N,), f32),
            pltpu.VMEM((N // NSUB,), f32),
            pltpu.VMEM((N,), f32),
            pltpu.VMEM((N,), f32),
            pltpu.VMEM((N,), f32),
            pltpu.VMEM((N,), f32),
            pltpu.VMEM((N,), f32),
            pltpu.VMEM((N,), f32),
            pltpu.VMEM((N,), f32),
            pltpu.VMEM((N,), f32),
            pltpu.VMEM((N,), f32),
            pltpu.VMEM((CHUNK,), jnp.int32),
            pltpu.VMEM((CHUNK,), jnp.int32),
            pltpu.VMEM((CHUNK,), f32),
            pltpu.VMEM((CHUNK,), jnp.int32),
            pltpu.VMEM((CHUNK,), jnp.int32),
            pltpu.VMEM((CHUNK,), f32),
            pltpu.SemaphoreType.DMA,
            pltpu.SemaphoreType.DMA,
            pltpu.SemaphoreType.DMA,
            pltpu.SemaphoreType.DMA,
            pltpu.SemaphoreType.DMA,
            pltpu.VMEM_SHARED((NSUB * N,), f32),
            pltpu.VMEM_SHARED((N,), f32),
            pltpu.VMEM_SHARED((E,), f32),
        ],
    )(ht, als, ald, px, py, pz, src, dst)


def kernel(x, edge_index, pos, mask, adjacency, W1, a1s, a1d, b1, g1, be1,
           W2, a2s, a2d, b2, g2, be2, Wf, bf):
    src = edge_index[0]
    dst = edge_index[1]
    px = pos[:, 0]
    py = pos[:, 1]
    pz = pos[:, 2]

    ht1, als1, ald1 = _tc_prep(x, W1, a1s, a1d)
    ot1 = _sc_layer(ht1, als1.reshape(N), ald1.reshape(N),
                    px, py, pz, src, dst)
    ht2, als2, ald2 = _tc_mid(ot1, b1.reshape(H, 1), g1.reshape(H, 1),
                              be1.reshape(H, 1), W2, a2s, a2d)
    ot2 = _sc_layer(ht2, als2.reshape(N), ald2.reshape(N),
                    px, py, pz, src, dst)
    out2d = _tc_final(ot2, b2.reshape(H, 1), g2.reshape(H, 1),
                      be2.reshape(H, 1), Wf, bf.reshape(1, D_OUT),
                      mask.reshape(N, 1), adjacency.reshape(N, D_OUT))
    return out2d.reshape(64, 40, 40)


# speedup vs baseline: 1.1441x; 1.1030x over previous
import jax
import jax.numpy as jnp
from jax import lax
from jax.experimental import pallas as pl
from jax.experimental.pallas import tpu as pltpu
from jax.experimental.pallas import tpu_sc as plsc

N = 2560
E = 40960
H = 128
HW = H // 2
D_OUT = 40

NCORE = 2
NSUB = 16
LANES = 16
CHUNK = 8192


def _pack_bf16_pairs(ht):
    lo16 = lax.bitcast_convert_type(ht[:HW].astype(jnp.bfloat16), jnp.uint16)
    hi16 = lax.bitcast_convert_type(ht[HW:].astype(jnp.bfloat16), jnp.uint16)
    return (hi16.astype(jnp.int32) << 16) | lo16.astype(jnp.int32)


def _prep_body(x_ref, w_ref, as_ref, ad_ref, htp_ref, als_ref, ald_ref):
    xv = x_ref[...]
    wv = w_ref[...]
    ht = lax.dot_general(wv, xv, (((0,), (1,)), ((), ())),
                         preferred_element_type=jnp.float32)
    htp_ref[...] = _pack_bf16_pairs(ht)
    als_ref[...] = lax.dot_general(as_ref[...], ht, (((1,), (0,)), ((), ())),
                                   preferred_element_type=jnp.float32)
    ald_ref[...] = lax.dot_general(ad_ref[...], ht, (((1,), (0,)), ((), ())),
                                   preferred_element_type=jnp.float32)


def _tc_prep(x, w, a_s, a_d):
    return pl.pallas_call(
        _prep_body,
        out_shape=(
            jax.ShapeDtypeStruct((HW, N), jnp.int32),
            jax.ShapeDtypeStruct((1, N), jnp.float32),
            jax.ShapeDtypeStruct((1, N), jnp.float32),
        ),
    )(x, w, a_s, a_d)


def _bn_relu(y, g, be):
    m = jnp.mean(y, axis=1, keepdims=True)
    c = y - m
    v = jnp.mean(c * c, axis=1, keepdims=True)
    return jnp.maximum(c * lax.rsqrt(v + 1e-5) * g + be, 0.0)


def _mid_body(ot_ref, b_ref, g_ref, be_ref, w_ref, as_ref, ad_ref,
              htp_ref, als_ref, ald_ref):
    y = ot_ref[...] + b_ref[...]
    h = _bn_relu(y, g_ref[...], be_ref[...])
    ht = lax.dot_general(w_ref[...], h, (((0,), (0,)), ((), ())),
                         preferred_element_type=jnp.float32)
    htp_ref[...] = _pack_bf16_pairs(ht)
    als_ref[...] = lax.dot_general(as_ref[...], ht, (((1,), (0,)), ((), ())),
                                   preferred_element_type=jnp.float32)
    ald_ref[...] = lax.dot_general(ad_ref[...], ht, (((1,), (0,)), ((), ())),
                                   preferred_element_type=jnp.float32)


def _tc_mid(ot, b, g, be, w, a_s, a_d):
    return pl.pallas_call(
        _mid_body,
        out_shape=(
            jax.ShapeDtypeStruct((HW, N), jnp.int32),
            jax.ShapeDtypeStruct((1, N), jnp.float32),
            jax.ShapeDtypeStruct((1, N), jnp.float32),
        ),
    )(ot, b, g, be, w, a_s, a_d)


def _final_body(ot_ref, b_ref, g_ref, be_ref, wf_ref, bf_ref, mask_ref,
                adj_ref, out_ref):
    y = ot_ref[...] + b_ref[...]
    h = _bn_relu(y, g_ref[...], be_ref[...])
    o = lax.dot_general(h, wf_ref[...], (((0,), (0,)), ((), ())),
                        preferred_element_type=jnp.float32)
    o = (o + bf_ref[...]) * mask_ref[...]
    out_ref[...] = o + adj_ref[...]


def _tc_final(ot, b, g, be, wf, bf, mask, adj2d):
    return pl.pallas_call(
        _final_body,
        out_shape=jax.ShapeDtypeStruct((N, D_OUT), jnp.float32),
    )(ot, b, g, be, wf, bf, mask, adj2d)


def _make_sc_body(first):
    def body(*refs):
        if first:
            (htp_hbm, als_hbm, ald_hbm, px_hbm, py_hbm, pz_hbm,
             src_hbm, dst_hbm, out_hbm, ew_hbm,
             srcv, dstv, alsv, aldv, pxv, pyv, pzv, ewb,
             exbuf, dpriv, denl, acc, tmpall,
             hw0, hw1, o0, o1, o2, o3,
             srcc, dstc, alpc, srcc2, dstc2, alpc2,
             sem0, sem1, sem2, sem3, sem4,
             sh_all, sh_den, sh_ex) = refs
        else:
            (htp_hbm, als_hbm, ald_hbm, ew_hbm,
             src_hbm, dst_hbm, out_hbm,
             srcv, dstv, alsv, aldv, pxv, pyv, pzv, ewb,
             exbuf, dpriv, denl, acc, tmpall,
             hw0, hw1, o0, o1, o2, o3,
             srcc, dstc, alpc, srcc2, dstc2, alpc2,
             sem0, sem1, sem2, sem3, sem4,
             sh_all, sh_den, sh_ex) = refs

        core = lax.axis_index("c")
        sub = lax.axis_index("s")
        tid = core * NSUB + sub
        eb = E // NSUB
        base = sub * eb
        cw = N // NSUB
        zero16 = jnp.zeros((LANES,), jnp.float32)
        w0r = tid * 2
        nch = E // CHUNK
        bufs = [(srcc, dstc, alpc), (srcc2, dstc2, alpc2)]

        pre2 = [
            pltpu.async_copy(htp_hbm.at[w0r], hw0, sem0),
            pltpu.async_copy(htp_hbm.at[w0r + 1], hw1, sem0),
            pltpu.async_copy(src_hbm.at[pl.ds(0, CHUNK)], srcc, sem0),
            pltpu.async_copy(dst_hbm.at[pl.ds(0, CHUNK)], dstc, sem0),
        ]
        scope1 = jax.named_scope("p1_edge")
        scope1.__enter__()
        pre1 = [
            pltpu.async_copy(src_hbm.at[pl.ds(base, eb)], srcv, sem1),
            pltpu.async_copy(dst_hbm.at[pl.ds(base, eb)], dstv, sem1),
            pltpu.async_copy(als_hbm, alsv, sem1),
            pltpu.async_copy(ald_hbm, aldv, sem1),
        ]
        if first:
            pre1 += [
                pltpu.async_copy(px_hbm, pxv, sem1),
                pltpu.async_copy(py_hbm, pyv, sem1),
                pltpu.async_copy(pz_hbm, pzv, sem1),
            ]
        else:
            pre1.append(
                pltpu.async_copy(ew_hbm.at[pl.ds(base, eb)], ewb, sem1))

        def zero_body(i, _):
            sl = pl.ds(i * LANES, LANES)
            dpriv[sl] = zero16
            o0[sl] = zero16
            o1[sl] = zero16
            o2[sl] = zero16
            o3[sl] = zero16
            return 0

        lax.fori_loop(0, N // LANES, zero_body, 0)
        for c in pre1:
            c.wait()

        U1 = 4

        def p1_step(off):
            sl = pl.ds(off, LANES)
            sv = srcv[sl]
            dv = dstv[sl]
            if first:
                ax = plsc.load_gather(pxv, [sv]) - plsc.load_gather(pxv, [dv])
                ay = plsc.load_gather(pyv, [sv]) - plsc.load_gather(pyv, [dv])
                az = plsc.load_gather(pzv, [sv]) - plsc.load_gather(pzv, [dv])
                s2 = ax * ax + ay * ay + az * az
                ib = jnp.int32(0x5F3759DF) - (plsc.bitcast(s2, jnp.int32) >> 1)
                y = plsc.bitcast(ib, jnp.float32)
                hs = 0.5 * s2
                y = y * (1.5 - (hs * y) * y)
                y = y * (1.5 - (hs * y) * y)
                y = y * (1.5 - (hs * y) * y)
                dist = s2 * y
                ew = 1.0 / (dist + 1e-6)
                ewb[sl] = ew
            else:
                ew = ewb[sl]
            e = plsc.load_gather(alsv, [sv]) + plsc.load_gather(aldv, [dv])
            e = jnp.maximum(e, 0.2 * e)
            return dv, jnp.exp(e) * ew

        def p1_body(i, _):
            off = i * (U1 * LANES)
            res = [p1_step(off + u * LANES) for u in range(U1)]
            for u in range(U1):
                dv, ex = res[u]
                exbuf[pl.ds(off + u * LANES, LANES)] = ex
                plsc.addupdate_scatter(dpriv, [dv], ex)
            return 0

        lax.fori_loop(0, eb // (U1 * LANES), p1_body, 0)
        if first:
            pltpu.sync_copy(ewb, ew_hbm.at[pl.ds(base, eb)])
        scope1.__exit__(None, None, None)

        scope2 = jax.named_scope("p1_combine")
        scope2.__enter__()
        pltpu.sync_copy(exbuf, sh_ex.at[pl.ds(base, eb)])
        pltpu.sync_copy(dpriv, sh_all.at[pl.ds(sub * N, N)])
        plsc.subcore_barrier()

        pending = [pltpu.async_copy(sh_ex.at[pl.ds(0, CHUNK)], alpc, sem2)]
        reads = [pltpu.async_copy(sh_all.at[pl.ds(r * N + sub * cw, cw)],
                                  tmpall.at[pl.ds(r * cw, cw)], sem3)
                 for r in range(NSUB)]
        for c in reads:
            c.wait()

        def comb_body(i, _):
            sl = pl.ds(i * LANES, LANES)
            s = tmpall[sl]
            for r in range(1, NSUB):
                s = s + tmpall[pl.ds(r * cw + i * LANES, LANES)]
            acc[sl] = s
            return 0

        lax.fori_loop(0, cw // LANES, comb_body, 0)
        pltpu.sync_copy(acc, sh_den.at[pl.ds(sub * cw, cw)])
        plsc.subcore_barrier()
        den_read = pltpu.async_copy(sh_den, denl, sem4)
        scope2.__exit__(None, None, None)

        scope3 = jax.named_scope("p2_agg")
        scope3.__enter__()
        for c in pre2:
            c.wait()

        def start_chunk(k, b):
            sb, db, ab = bufs[b]
            c1 = pltpu.async_copy(src_hbm.at[pl.ds(k * CHUNK, CHUNK)], sb,
                                  sem0)
            c2 = pltpu.async_copy(dst_hbm.at[pl.ds(k * CHUNK, CHUNK)], db,
                                  sem1)
            c3 = pltpu.async_copy(sh_ex.at[pl.ds(k * CHUNK, CHUNK)], ab,
                                  sem2)
            return (c1, c2, c3)

        U = 4
        himask = jnp.int32(-65536)

        for k in range(nch):
            for c in pending:
                c.wait()
            if k + 1 < nch:
                pending = start_chunk(k + 1, (k + 1) % 2)
            sb, db, ab = bufs[k % 2]

            def p2_body(i, _):
                off = i * (U * LANES)
                sls = [pl.ds(off + u * LANES, LANES) for u in range(U)]
                svs = [sb[sls[u]] for u in range(U)]
                dvs = [db[sls[u]] for u in range(U)]
                avs = [ab[sls[u]] for u in range(U)]
                ws = [(plsc.load_gather(hw0, [svs[u]]),
                       plsc.load_gather(hw1, [svs[u]])) for u in range(U)]
                ms = []
                for u in range(U):
                    wa, wb = ws[u]
                    ms.append((
                        plsc.bitcast(wa << 16, jnp.float32) * avs[u],
                        plsc.bitcast(wa & himask, jnp.float32) * avs[u],
                        plsc.bitcast(wb << 16, jnp.float32) * avs[u],
                        plsc.bitcast(wb & himask, jnp.float32) * avs[u],
                    ))
                for u in range(U):
                    plsc.addupdate_scatter(o0, [dvs[u]], ms[u][0])
                    plsc.addupdate_scatter(o1, [dvs[u]], ms[u][1])
                    plsc.addupdate_scatter(o2, [dvs[u]], ms[u][2])
                    plsc.addupdate_scatter(o3, [dvs[u]], ms[u][3])
                return 0

            lax.fori_loop(0, CHUNK // (U * LANES), p2_body, 0)

        den_read.wait()

        def norm_body(i, _):
            sl = pl.ds(i * LANES, LANES)
            r = 1.0 / (denl[sl] + 1e-16)
            o0[sl] = o0[sl] * r
            o1[sl] = o1[sl] * r
            o2[sl] = o2[sl] * r
            o3[sl] = o3[sl] * r
            return 0

        lax.fori_loop(0, N // LANES, norm_body, 0)

        pltpu.sync_copy(o0, out_hbm.at[w0r])
        pltpu.sync_copy(o1, out_hbm.at[w0r + HW])
        pltpu.sync_copy(o2, out_hbm.at[w0r + 1])
        pltpu.sync_copy(o3, out_hbm.at[w0r + 1 + HW])
        scope3.__exit__(None, None, None)

    return body


_SC_SCRATCH = [
    pltpu.VMEM((E // NSUB,), jnp.int32),
    pltpu.VMEM((E // NSUB,), jnp.int32),
    pltpu.VMEM((N,), jnp.float32),
    pltpu.VMEM((N,), jnp.float32),
    pltpu.VMEM((N,), jnp.float32),
    pltpu.VMEM((N,), jnp.float32),
    pltpu.VMEM((N,), jnp.float32),
    pltpu.VMEM((E // NSUB,), jnp.float32),
    pltpu.VMEM((E // NSUB,), jnp.float32),
    pltpu.VMEM((N,), jnp.float32),
    pltpu.VMEM((N,), jnp.float32),
    pltpu.VMEM((N // NSUB,), jnp.float32),
    pltpu.VMEM((N,), jnp.float32),
    pltpu.VMEM((N,), jnp.int32),
    pltpu.VMEM((N,), jnp.int32),
    pltpu.VMEM((N,), jnp.float32),
    pltpu.VMEM((N,), jnp.float32),
    pltpu.VMEM((N,), jnp.float32),
    pltpu.VMEM((N,), jnp.float32),
    pltpu.VMEM((CHUNK,), jnp.int32),
    pltpu.VMEM((CHUNK,), jnp.int32),
    pltpu.VMEM((CHUNK,), jnp.float32),
    pltpu.VMEM((CHUNK,), jnp.int32),
    pltpu.VMEM((CHUNK,), jnp.int32),
    pltpu.VMEM((CHUNK,), jnp.float32),
    pltpu.SemaphoreType.DMA,
    pltpu.SemaphoreType.DMA,
    pltpu.SemaphoreType.DMA,
    pltpu.SemaphoreType.DMA,
    pltpu.SemaphoreType.DMA,
    pltpu.VMEM_SHARED((NSUB * N,), jnp.float32),
    pltpu.VMEM_SHARED((N,), jnp.float32),
    pltpu.VMEM_SHARED((E,), jnp.float32),
]


def _sc_mesh():
    return plsc.VectorSubcoreMesh(core_axis_name="c", subcore_axis_name="s",
                                  num_cores=NCORE, num_subcores=NSUB)


def _sc_layer1(htp, als, ald, px, py, pz, src, dst):
    return pl.kernel(
        _make_sc_body(True),
        out_type=(jax.ShapeDtypeStruct((H, N), jnp.float32),
                  jax.ShapeDtypeStruct((E,), jnp.float32)),
        mesh=_sc_mesh(),
        compiler_params=pltpu.CompilerParams(needs_layout_passes=False),
        scratch_types=_SC_SCRATCH,
    )(htp, als, ald, px, py, pz, src, dst)


def _sc_layer2(htp, als, ald, ew, src, dst):
    return pl.kernel(
        _make_sc_body(False),
        out_type=jax.ShapeDtypeStruct((H, N), jnp.float32),
        mesh=_sc_mesh(),
        compiler_params=pltpu.CompilerParams(needs_layout_passes=False),
        scratch_types=_SC_SCRATCH,
    )(htp, als, ald, ew, src, dst)


def kernel(x, edge_index, pos, mask, adjacency, W1, a1s, a1d, b1, g1, be1,
           W2, a2s, a2d, b2, g2, be2, Wf, bf):
    src = edge_index[0]
    dst = edge_index[1]
    px = pos[:, 0]
    py = pos[:, 1]
    pz = pos[:, 2]

    htp1, als1, ald1 = _tc_prep(x, W1, a1s, a1d)
    ot1, ew = _sc_layer1(htp1, als1.reshape(N), ald1.reshape(N),
                         px, py, pz, src, dst)
    htp2, als2, ald2 = _tc_mid(ot1, b1.reshape(H, 1), g1.reshape(H, 1),
                               be1.reshape(H, 1), W2, a2s, a2d)
    ot2 = _sc_layer2(htp2, als2.reshape(N), ald2.reshape(N), ew, src, dst)
    out2d = _tc_final(ot2, b2.reshape(H, 1), g2.reshape(H, 1),
                      be2.reshape(H, 1), Wf, bf.reshape(1, D_OUT),
                      mask.reshape(N, 1), adjacency.reshape(N, D_OUT))
    return out2d.reshape(64, 40, 40)


# speedup vs baseline: 1.1820x; 1.0331x over previous
import jax
import jax.numpy as jnp
from jax import lax
from jax.experimental import pallas as pl
from jax.experimental.pallas import tpu as pltpu
from jax.experimental.pallas import tpu_sc as plsc

N = 2560
E = 40960
H = 128
HW = H // 2
D_OUT = 40

NCORE = 2
NSUB = 16
LANES = 16
CHUNK = 8192


def _pack_bf16_pairs(ht):
    lo16 = lax.bitcast_convert_type(ht[:HW].astype(jnp.bfloat16), jnp.uint16)
    hi16 = lax.bitcast_convert_type(ht[HW:].astype(jnp.bfloat16), jnp.uint16)
    return (hi16.astype(jnp.int32) << 16) | lo16.astype(jnp.int32)


def _prep_body(x_ref, w_ref, as_ref, ad_ref, htp_ref, als_ref, ald_ref):
    xv = x_ref[...]
    wv = w_ref[...]
    ht = lax.dot_general(wv, xv, (((0,), (1,)), ((), ())),
                         preferred_element_type=jnp.float32)
    htp_ref[...] = _pack_bf16_pairs(ht)
    als_ref[...] = lax.dot_general(as_ref[...], ht, (((1,), (0,)), ((), ())),
                                   preferred_element_type=jnp.float32)
    ald_ref[...] = lax.dot_general(ad_ref[...], ht, (((1,), (0,)), ((), ())),
                                   preferred_element_type=jnp.float32)


def _tc_prep(x, w, a_s, a_d):
    return pl.pallas_call(
        _prep_body,
        out_shape=(
            jax.ShapeDtypeStruct((HW, N), jnp.int32),
            jax.ShapeDtypeStruct((1, N), jnp.float32),
            jax.ShapeDtypeStruct((1, N), jnp.float32),
        ),
    )(x, w, a_s, a_d)


def _bn_relu(y, g, be):
    m = jnp.mean(y, axis=1, keepdims=True)
    c = y - m
    v = jnp.mean(c * c, axis=1, keepdims=True)
    return jnp.maximum(c * lax.rsqrt(v + 1e-5) * g + be, 0.0)


def _mid_body(ot_ref, b_ref, g_ref, be_ref, w_ref, as_ref, ad_ref,
              htp_ref, als_ref, ald_ref):
    y = ot_ref[...] + b_ref[...]
    h = _bn_relu(y, g_ref[...], be_ref[...])
    ht = lax.dot_general(w_ref[...], h, (((0,), (0,)), ((), ())),
                         preferred_element_type=jnp.float32)
    htp_ref[...] = _pack_bf16_pairs(ht)
    als_ref[...] = lax.dot_general(as_ref[...], ht, (((1,), (0,)), ((), ())),
                                   preferred_element_type=jnp.float32)
    ald_ref[...] = lax.dot_general(ad_ref[...], ht, (((1,), (0,)), ((), ())),
                                   preferred_element_type=jnp.float32)


def _tc_mid(ot, b, g, be, w, a_s, a_d):
    return pl.pallas_call(
        _mid_body,
        out_shape=(
            jax.ShapeDtypeStruct((HW, N), jnp.int32),
            jax.ShapeDtypeStruct((1, N), jnp.float32),
            jax.ShapeDtypeStruct((1, N), jnp.float32),
        ),
    )(ot, b, g, be, w, a_s, a_d)


def _final_body(ot_ref, b_ref, g_ref, be_ref, wf_ref, bf_ref, mask_ref,
                adj_ref, out_ref):
    y = ot_ref[...] + b_ref[...]
    h = _bn_relu(y, g_ref[...], be_ref[...])
    o = lax.dot_general(h, wf_ref[...], (((0,), (0,)), ((), ())),
                        preferred_element_type=jnp.float32)
    o = (o + bf_ref[...]) * mask_ref[...]
    out_ref[...] = o + adj_ref[...]


def _tc_final(ot, b, g, be, wf, bf, mask, adj2d):
    return pl.pallas_call(
        _final_body,
        out_shape=jax.ShapeDtypeStruct((N, D_OUT), jnp.float32),
    )(ot, b, g, be, wf, bf, mask, adj2d)


def _make_sc_body(first):
    def body(*refs):
        if first:
            (htp_hbm, als_hbm, ald_hbm, px_hbm, py_hbm, pz_hbm,
             src_hbm, dst_hbm, out_hbm, ew_hbm,
             srcv, dstv, alsv, aldv, pxv, pyv, pzv, ewb,
             exbuf, dpriv, denl, acc, tmpall,
             hw0, hw1, o0, o1, o2, o3,
             srcc, dstc, alpc, srcc2, dstc2, alpc2,
             sem0, sem1, sem2, sem3, sem4,
             sh_all, sh_den, sh_ex) = refs
        else:
            (htp_hbm, als_hbm, ald_hbm, ew_hbm,
             src_hbm, dst_hbm, out_hbm,
             srcv, dstv, alsv, aldv, pxv, pyv, pzv, ewb,
             exbuf, dpriv, denl, acc, tmpall,
             hw0, hw1, o0, o1, o2, o3,
             srcc, dstc, alpc, srcc2, dstc2, alpc2,
             sem0, sem1, sem2, sem3, sem4,
             sh_all, sh_den, sh_ex) = refs

        core = lax.axis_index("c")
        sub = lax.axis_index("s")
        tid = core * NSUB + sub
        eb = E // NSUB
        base = sub * eb
        cw = N // NSUB
        zero16 = jnp.zeros((LANES,), jnp.float32)
        w0r = tid * 2
        nch = E // CHUNK
        bufs = [(srcc, dstc, alpc), (srcc2, dstc2, alpc2)]

        pre2 = [
            pltpu.async_copy(htp_hbm.at[w0r], hw0, sem0),
            pltpu.async_copy(htp_hbm.at[w0r + 1], hw1, sem0),
            pltpu.async_copy(src_hbm.at[pl.ds(0, CHUNK)], srcc, sem0),
            pltpu.async_copy(dst_hbm.at[pl.ds(0, CHUNK)], dstc, sem0),
        ]
        scope1 = jax.named_scope("p1_edge")
        scope1.__enter__()
        pre1 = [
            pltpu.async_copy(src_hbm.at[pl.ds(base, eb)], srcv, sem1),
            pltpu.async_copy(dst_hbm.at[pl.ds(base, eb)], dstv, sem1),
            pltpu.async_copy(als_hbm.at[0], alsv, sem1),
            pltpu.async_copy(ald_hbm.at[0], aldv, sem1),
        ]
        if first:
            pre1 += [
                pltpu.async_copy(px_hbm, pxv, sem1),
                pltpu.async_copy(py_hbm, pyv, sem1),
                pltpu.async_copy(pz_hbm, pzv, sem1),
            ]
        else:
            pre1.append(
                pltpu.async_copy(ew_hbm.at[pl.ds(base, eb)], ewb, sem1))

        def zero_body(i, _):
            sl = pl.ds(i * LANES, LANES)
            dpriv[sl] = zero16
            o0[sl] = zero16
            o1[sl] = zero16
            o2[sl] = zero16
            o3[sl] = zero16
            return 0

        lax.fori_loop(0, N // LANES, zero_body, 0)
        for c in pre1:
            c.wait()

        U1 = 4

        def p1_step(off):
            sl = pl.ds(off, LANES)
            sv = srcv[sl]
            dv = dstv[sl]
            if first:
                ax = plsc.load_gather(pxv, [sv]) - plsc.load_gather(pxv, [dv])
                ay = plsc.load_gather(pyv, [sv]) - plsc.load_gather(pyv, [dv])
                az = plsc.load_gather(pzv, [sv]) - plsc.load_gather(pzv, [dv])
                s2 = ax * ax + ay * ay + az * az
                ib = jnp.int32(0x5F3759DF) - (plsc.bitcast(s2, jnp.int32) >> 1)
                y = plsc.bitcast(ib, jnp.float32)
                hs = 0.5 * s2
                y = y * (1.5 - (hs * y) * y)
                y = y * (1.5 - (hs * y) * y)
                dist = s2 * y
                ew = 1.0 / (dist + 1e-6)
                ewb[sl] = ew
            else:
                ew = ewb[sl]
            e = plsc.load_gather(alsv, [sv]) + plsc.load_gather(aldv, [dv])
            e = jnp.maximum(e, 0.2 * e)
            return dv, jnp.exp(e) * ew

        def p1_body(i, _):
            off = i * (U1 * LANES)
            res = [p1_step(off + u * LANES) for u in range(U1)]
            for u in range(U1):
                dv, ex = res[u]
                exbuf[pl.ds(off + u * LANES, LANES)] = ex
                plsc.addupdate_scatter(dpriv, [dv], ex)
            return 0

        lax.fori_loop(0, eb // (U1 * LANES), p1_body, 0)
        ew_write = None
        if first:
            ew_write = pltpu.async_copy(ewb, ew_hbm.at[pl.ds(base, eb)],
                                        sem4)
        scope1.__exit__(None, None, None)

        scope2 = jax.named_scope("p1_combine")
        scope2.__enter__()
        pltpu.sync_copy(exbuf, sh_ex.at[pl.ds(base, eb)])
        pltpu.sync_copy(dpriv, sh_all.at[pl.ds(sub * N, N)])
        plsc.subcore_barrier()

        pending = [pltpu.async_copy(sh_ex.at[pl.ds(0, CHUNK)], alpc, sem2)]
        reads = [pltpu.async_copy(sh_all.at[pl.ds(r * N + sub * cw, cw)],
                                  tmpall.at[pl.ds(r * cw, cw)], sem3)
                 for r in range(NSUB)]
        for c in reads:
            c.wait()

        def comb_body(i, _):
            sl = pl.ds(i * LANES, LANES)
            s = tmpall[sl]
            for r in range(1, NSUB):
                s = s + tmpall[pl.ds(r * cw + i * LANES, LANES)]
            acc[sl] = s
            return 0

        lax.fori_loop(0, cw // LANES, comb_body, 0)
        pltpu.sync_copy(acc, sh_den.at[pl.ds(sub * cw, cw)])
        plsc.subcore_barrier()
        if ew_write is not None:
            ew_write.wait()
        den_read = pltpu.async_copy(sh_den, denl, sem4)
        scope2.__exit__(None, None, None)

        scope3 = jax.named_scope("p2_agg")
        scope3.__enter__()
        for c in pre2:
            c.wait()

        def start_chunk(k, b):
            sb, db, ab = bufs[b]
            c1 = pltpu.async_copy(src_hbm.at[pl.ds(k * CHUNK, CHUNK)], sb,
                                  sem0)
            c2 = pltpu.async_copy(dst_hbm.at[pl.ds(k * CHUNK, CHUNK)], db,
                                  sem1)
            c3 = pltpu.async_copy(sh_ex.at[pl.ds(k * CHUNK, CHUNK)], ab,
                                  sem2)
            return (c1, c2, c3)

        U = 4
        himask = jnp.int32(-65536)

        for k in range(nch):
            for c in pending:
                c.wait()
            if k + 1 < nch:
                pending = start_chunk(k + 1, (k + 1) % 2)
            sb, db, ab = bufs[k % 2]

            def p2_body(i, _):
                off = i * (U * LANES)
                sls = [pl.ds(off + u * LANES, LANES) for u in range(U)]
                svs = [sb[sls[u]] for u in range(U)]
                dvs = [db[sls[u]] for u in range(U)]
                avs = [ab[sls[u]] for u in range(U)]
                ws = [(plsc.load_gather(hw0, [svs[u]]),
                       plsc.load_gather(hw1, [svs[u]])) for u in range(U)]
                ms = []
                for u in range(U):
                    wa, wb = ws[u]
                    ms.append((
                        plsc.bitcast(wa << 16, jnp.float32) * avs[u],
                        plsc.bitcast(wa & himask, jnp.float32) * avs[u],
                        plsc.bitcast(wb << 16, jnp.float32) * avs[u],
                        plsc.bitcast(wb & himask, jnp.float32) * avs[u],
                    ))
                for u in range(U):
                    plsc.addupdate_scatter(o0, [dvs[u]], ms[u][0])
                    plsc.addupdate_scatter(o1, [dvs[u]], ms[u][1])
                    plsc.addupdate_scatter(o2, [dvs[u]], ms[u][2])
                    plsc.addupdate_scatter(o3, [dvs[u]], ms[u][3])
                return 0

            lax.fori_loop(0, CHUNK // (U * LANES), p2_body, 0)

        den_read.wait()

        def norm_body(i, _):
            sl = pl.ds(i * LANES, LANES)
            r = 1.0 / (denl[sl] + 1e-16)
            o0[sl] = o0[sl] * r
            o1[sl] = o1[sl] * r
            o2[sl] = o2[sl] * r
            o3[sl] = o3[sl] * r
            return 0

        lax.fori_loop(0, N // LANES, norm_body, 0)

        pltpu.sync_copy(o0, out_hbm.at[w0r])
        pltpu.sync_copy(o1, out_hbm.at[w0r + HW])
        pltpu.sync_copy(o2, out_hbm.at[w0r + 1])
        pltpu.sync_copy(o3, out_hbm.at[w0r + 1 + HW])
        scope3.__exit__(None, None, None)

    return body


_SC_SCRATCH = [
    pltpu.VMEM((E // NSUB,), jnp.int32),
    pltpu.VMEM((E // NSUB,), jnp.int32),
    pltpu.VMEM((N,), jnp.float32),
    pltpu.VMEM((N,), jnp.float32),
    pltpu.VMEM((N,), jnp.float32),
    pltpu.VMEM((N,), jnp.float32),
    pltpu.VMEM((N,), jnp.float32),
    pltpu.VMEM((E // NSUB,), jnp.float32),
    pltpu.VMEM((E // NSUB,), jnp.float32),
    pltpu.VMEM((N,), jnp.float32),
    pltpu.VMEM((N,), jnp.float32),
    pltpu.VMEM((N // NSUB,), jnp.float32),
    pltpu.VMEM((N,), jnp.float32),
    pltpu.VMEM((N,), jnp.int32),
    pltpu.VMEM((N,), jnp.int32),
    pltpu.VMEM((N,), jnp.float32),
    pltpu.VMEM((N,), jnp.float32),
    pltpu.VMEM((N,), jnp.float32),
    pltpu.VMEM((N,), jnp.float32),
    pltpu.VMEM((CHUNK,), jnp.int32),
    pltpu.VMEM((CHUNK,), jnp.int32),
    pltpu.VMEM((CHUNK,), jnp.float32),
    pltpu.VMEM((CHUNK,), jnp.int32),
    pltpu.VMEM((CHUNK,), jnp.int32),
    pltpu.VMEM((CHUNK,), jnp.float32),
    pltpu.SemaphoreType.DMA,
    pltpu.SemaphoreType.DMA,
    pltpu.SemaphoreType.DMA,
    pltpu.SemaphoreType.DMA,
    pltpu.SemaphoreType.DMA,
    pltpu.VMEM_SHARED((NSUB * N,), jnp.float32),
    pltpu.VMEM_SHARED((N,), jnp.float32),
    pltpu.VMEM_SHARED((E,), jnp.float32),
]


def _sc_mesh():
    return plsc.VectorSubcoreMesh(core_axis_name="c", subcore_axis_name="s",
                                  num_cores=NCORE, num_subcores=NSUB)


def _sc_layer1(htp, als, ald, px, py, pz, src, dst):
    return pl.kernel(
        _make_sc_body(True),
        out_type=(jax.ShapeDtypeStruct((H, N), jnp.float32),
                  jax.ShapeDtypeStruct((E,), jnp.float32)),
        mesh=_sc_mesh(),
        compiler_params=pltpu.CompilerParams(needs_layout_passes=False),
        scratch_types=_SC_SCRATCH,
    )(htp, als, ald, px, py, pz, src, dst)


def _sc_layer2(htp, als, ald, ew, src, dst):
    return pl.kernel(
        _make_sc_body(False),
        out_type=jax.ShapeDtypeStruct((H, N), jnp.float32),
        mesh=_sc_mesh(),
        compiler_params=pltpu.CompilerParams(needs_layout_passes=False),
        scratch_types=_SC_SCRATCH,
    )(htp, als, ald, ew, src, dst)


def kernel(x, edge_index, pos, mask, adjacency, W1, a1s, a1d, b1, g1, be1,
           W2, a2s, a2d, b2, g2, be2, Wf, bf):
    src = edge_index[0]
    dst = edge_index[1]
    px = pos[:, 0]
    py = pos[:, 1]
    pz = pos[:, 2]

    htp1, als1, ald1 = _tc_prep(x, W1, a1s, a1d)
    ot1, ew = _sc_layer1(htp1, als1, ald1, px, py, pz, src, dst)
    htp2, als2, ald2 = _tc_mid(ot1, b1.reshape(H, 1), g1.reshape(H, 1),
                               be1.reshape(H, 1), W2, a2s, a2d)
    ot2 = _sc_layer2(htp2, als2, ald2, ew, src, dst)
    out2d = _tc_final(ot2, b2.reshape(H, 1), g2.reshape(H, 1),
                      be2.reshape(H, 1), Wf, bf.reshape(1, D_OUT),
                      mask.reshape(N, 1), adjacency.reshape(N, D_OUT))
    return out2d.reshape(64, 40, 40)
